# Initial kernel scaffold; baseline (speedup 1.0000x reference)
import jax
import jax.numpy as jnp
from jax.experimental import pallas as pl


def kernel(x, W):
    raise NotImplementedError("write your pallas kernel here")


# speedup vs baseline: 2.5483x; 2.5483x over previous
import functools

import jax
import jax.numpy as jnp
from jax import lax
from jax.experimental import pallas as pl
from jax.experimental.pallas import tpu as pltpu
from jax.experimental.pallas import tpu_sc as plsc

B, C, N = 4, 128, 1024
K = 20
OUT = 256
KP = 32
NEG = -3.0e38

NW = 32
R = B * N
PW = R // NW
CH = 4
NCH = PW // CH


def _prep_body(xt_ref, w_ref, y1_ref, z_ref, idx_ref):
    b = pl.program_id(0)
    xt = xt_ref[...]
    dn = (((1,), (1,)), ((), ()))
    g = lax.dot_general(xt, xt, dn,
                        preferred_element_type=jnp.float32)
    xx = jnp.sum(xt * xt, axis=1, keepdims=True)
    d = (2.0 * g - xx) - jnp.transpose(xx)

    col = lax.broadcasted_iota(jnp.int32, (N, N), 1)
    lanek = lax.broadcasted_iota(jnp.int32, (N, KP), 1)
    base = b * N
    idxacc = jnp.full((N, KP), base, jnp.int32)
    for k in range(K):
        m = jnp.max(d, axis=1, keepdims=True)
        cand = jnp.where(d == m, col, N)
        am = jnp.min(cand, axis=1, keepdims=True)
        idxacc = jnp.where(lanek == k, am + base, idxacc)
        d = jnp.where(col == am, NEG, d)
    idx_ref[...] = idxacc

    w1 = w_ref[:, :C]
    wd = w_ref[:, C:] - w1
    y1_ref[...] = lax.dot_general(xt, w1, dn,
                                  preferred_element_type=jnp.float32,
                                  precision=lax.Precision.HIGHEST)
    z_ref[...] = lax.dot_general(xt, wd, dn,
                                 preferred_element_type=jnp.float32,
                                 precision=lax.Precision.HIGHEST)


def _prep(xt, w):
    return pl.pallas_call(
        _prep_body,
        grid=(B,),
        in_specs=[
            pl.BlockSpec((None, N, C), lambda b: (b, 0, 0)),
            pl.BlockSpec((OUT, 2 * C), lambda b: (0, 0)),
        ],
        out_specs=[
            pl.BlockSpec((None, N, OUT), lambda b: (b, 0, 0)),
            pl.BlockSpec((None, N, OUT), lambda b: (b, 0, 0)),
            pl.BlockSpec((None, N, KP), lambda b: (b, 0, 0)),
        ],
        out_shape=[
            jax.ShapeDtypeStruct((B, N, OUT), jnp.float32),
            jax.ShapeDtypeStruct((B, N, OUT), jnp.float32),
            jax.ShapeDtypeStruct((B, N, KP), jnp.int32),
        ],
    )(xt, w)


def _gather_max(y1f, idxf, zf):
    mesh = plsc.VectorSubcoreMesh(core_axis_name="c", subcore_axis_name="s")

    @functools.partial(
        pl.kernel,
        out_type=jax.ShapeDtypeStruct((R, OUT), jnp.float32),
        mesh=mesh,
        scratch_types=[
            pltpu.VMEM((CH * KP,), jnp.int32),
            pltpu.VMEM((CH * KP, OUT), jnp.float32),
            pltpu.VMEM((CH, OUT), jnp.float32),
            pltpu.VMEM((CH, OUT), jnp.float32),
            pltpu.SemaphoreType.DMA,
        ],
    )
    def body(y1_hbm, idx_hbm, z_hbm, out_hbm, idx_v, rows_v, z_v, out_v, sem):
        wid = lax.axis_index("s") * 2 + lax.axis_index("c")
        row0w = wid * PW

        def chunk_body(ch, carry):
            row0 = row0w + ch * CH
            pltpu.sync_copy(idx_hbm.at[pl.ds(row0 * KP, CH * KP)], idx_v)
            pltpu.sync_copy(z_hbm.at[pl.ds(row0, CH)], z_v)
            pltpu.async_copy(y1_hbm.at[idx_v], rows_v, sem).wait()

            def pair_body(p, carry2):
                for c in range(OUT // 16):
                    sl = pl.ds(c * 16, 16)
                    acc = rows_v[p * KP, sl]
                    for kk in range(1, K):
                        acc = jnp.maximum(acc, rows_v[p * KP + kk, sl])
                    out_v[p, sl] = acc + z_v[p, sl]
                return carry2

            lax.fori_loop(0, CH, pair_body, 0)
            pltpu.sync_copy(out_v, out_hbm.at[pl.ds(row0, CH)])
            return carry

        lax.fori_loop(0, NCH, chunk_body, 0)

    return body(y1f, idxf, zf)


def kernel(x, W):
    xt = jnp.transpose(x, (0, 2, 1))
    y1, z, idx = _prep(xt, W)
    outf = _gather_max(y1.reshape(R, OUT),
                       idx.reshape(R * KP),
                       z.reshape(R, OUT))
    return jnp.transpose(outf.reshape(B, N, OUT), (0, 2, 1))


# speedup vs baseline: 9.4509x; 3.7087x over previous
import functools

import jax
import jax.numpy as jnp
from jax import lax
from jax.experimental import pallas as pl
from jax.experimental.pallas import tpu as pltpu
from jax.experimental.pallas import tpu_sc as plsc

B, C, N = 4, 128, 1024
K = 20
OUT = 256
KP = 32
NEG = -3.0e38

NW = 32
R = B * N
PW = R // NW
CH = 4
NCH = PW // CH


def _prep_body(xt_ref, w_ref, y1_ref, z_ref, idx_ref):
    b = pl.program_id(0)
    xt = xt_ref[...]
    dn = (((1,), (1,)), ((), ()))
    g = lax.dot_general(xt, xt, dn,
                        preferred_element_type=jnp.float32)
    xx = jnp.sum(xt * xt, axis=1, keepdims=True)
    d = (2.0 * g - xx) - jnp.transpose(xx)

    col = lax.broadcasted_iota(jnp.int32, (N, N), 1)
    lanek = lax.broadcasted_iota(jnp.int32, (N, KP), 1)
    base = b * N
    idxacc = jnp.full((N, KP), base, jnp.int32)
    for k in range(K):
        m = jnp.max(d, axis=1, keepdims=True)
        cand = jnp.where(d == m, col, N)
        am = jnp.min(cand, axis=1, keepdims=True)
        idxacc = jnp.where(lanek == k, am + base, idxacc)
        d = jnp.where(col == am, NEG, d)
    idx_ref[...] = idxacc

    w1 = w_ref[:, :C]
    wd = w_ref[:, C:] - w1
    y1_ref[...] = lax.dot_general(xt, w1, dn,
                                  preferred_element_type=jnp.float32,
                                  precision=lax.Precision.HIGHEST)
    z_ref[...] = lax.dot_general(xt, wd, dn,
                                 preferred_element_type=jnp.float32,
                                 precision=lax.Precision.HIGHEST)


def _prep(xt, w):
    return pl.pallas_call(
        _prep_body,
        grid=(B,),
        in_specs=[
            pl.BlockSpec((None, N, C), lambda b: (b, 0, 0)),
            pl.BlockSpec((OUT, 2 * C), lambda b: (0, 0)),
        ],
        out_specs=[
            pl.BlockSpec((None, N, OUT), lambda b: (b, 0, 0)),
            pl.BlockSpec((None, N, OUT), lambda b: (b, 0, 0)),
            pl.BlockSpec((None, N, KP), lambda b: (b, 0, 0)),
        ],
        out_shape=[
            jax.ShapeDtypeStruct((B, N, OUT), jnp.float32),
            jax.ShapeDtypeStruct((B, N, OUT), jnp.float32),
            jax.ShapeDtypeStruct((B, N, KP), jnp.int32),
        ],
    )(xt, w)


def _gather_max(y1f, idxf, zf):
    mesh = plsc.VectorSubcoreMesh(core_axis_name="c", subcore_axis_name="s")

    @functools.partial(
        pl.kernel,
        out_type=jax.ShapeDtypeStruct((R, OUT), jnp.float32),
        mesh=mesh,
        scratch_types=[
            pltpu.VMEM((PW * K,), jnp.int32),
            pltpu.VMEM((PW, OUT), jnp.float32),
            pltpu.VMEM((CH * K, OUT), jnp.float32),
            pltpu.VMEM((CH * K, OUT), jnp.float32),
            pltpu.SemaphoreType.DMA,
            pltpu.SemaphoreType.DMA,
        ],
    )
    def body(y1_hbm, idx_hbm, z_hbm, out_hbm,
             idx_v, out_v, rows0, rows1, sem0, sem1):
        wid = lax.axis_index("s") * 2 + lax.axis_index("c")
        row0w = wid * PW
        pltpu.sync_copy(idx_hbm.at[pl.ds(row0w * K, PW * K)], idx_v)
        pltpu.sync_copy(z_hbm.at[pl.ds(row0w, PW)], out_v)

        def gather(ch, rows, sem):
            return pltpu.async_copy(
                y1_hbm.at[idx_v.at[pl.ds(ch * CH * K, CH * K)]], rows, sem)

        def compute(ch, rows):
            def pair_body(p, carry):
                for c in range(OUT // 16):
                    sl = pl.ds(c * 16, 16)
                    acc = rows[p * K, sl]
                    for kk in range(1, K):
                        acc = jnp.maximum(acc, rows[p * K + kk, sl])
                    q = ch * CH + p
                    out_v[q, sl] = out_v[q, sl] + acc
                return carry
            lax.fori_loop(0, CH, pair_body, 0)

        gather(0, rows0, sem0)

        def two_chunks(i, carry):
            ch0 = i * 2
            nxt = gather(ch0 + 1, rows1, sem1)
            cp_wait(sem0, rows0)
            compute(ch0, rows0)
            @pl.when(i < NCH // 2 - 1)
            def _():
                gather(ch0 + 2, rows0, sem0)
            cp_wait(sem1, rows1)
            compute(ch0 + 1, rows1)
            return carry

        def cp_wait(sem, rows):
            pltpu.make_async_copy(y1_hbm.at[idx_v.at[pl.ds(0, CH * K)]],
                                  rows, sem).wait()

        lax.fori_loop(0, NCH // 2, two_chunks, 0)
        pltpu.sync_copy(out_v, out_hbm.at[pl.ds(row0w, PW)])

    return body(y1f, idxf, zf)


def kernel(x, W):
    xt = jnp.transpose(x, (0, 2, 1))
    y1, z, idx = _prep(xt, W)
    outf = _gather_max(y1.reshape(R, OUT),
                       idx[:, :, :K].reshape(R * K),
                       z.reshape(R, OUT))
    return jnp.transpose(outf.reshape(B, N, OUT), (0, 2, 1))
